# baseline (speedup 1.0000x reference)
import functools

import jax
import jax.numpy as jnp
from jax import lax
from jax.experimental import pallas as pl
from jax.experimental.pallas import tpu as pltpu
from jax.experimental.pallas import tpu_sc as plsc

H = 16
S = 2048
TBL = 8191
COPY = 4096
NW = 32


_mesh = plsc.VectorSubcoreMesh(core_axis_name="c", subcore_axis_name="s")


def _make(hk):
    wph = NW // hk
    ncls = 16 // wph
    npair = ncls // 2

    @functools.partial(
        pl.kernel,
        out_type=jax.ShapeDtypeStruct((hk, S, S), jnp.float32),
        mesh=_mesh,
        scratch_types=[
            pltpu.VMEM((TBL,), jnp.float32),
            pltpu.VMEM((2, 8, COPY), jnp.float32),
            pltpu.SemaphoreType.DMA,
        ],
    )
    def _rel_pos_bias(tbl_hbm, out_hbm, t_v, tab_v, sem):
        cid = lax.axis_index("c")
        sid = lax.axis_index("s")
        wid = sid * 2 + cid
        h = wid // wph
        part = wid % wph

        pltpu.sync_copy(tbl_hbm.at[h], t_v)

        lanes = lax.iota(jnp.int32, 16)

        def lane_perm(v, idx):
            dnums = lax.GatherDimensionNumbers(
                offset_dims=(), collapsed_slice_dims=(0,), start_index_map=(0,)
            )
            return lax.gather(
                v, idx[:, None], dnums, (1,),
                mode=lax.GatherScatterMode.PROMISE_IN_BOUNDS,
            )

        def drain_one(buf):
            pltpu.make_async_copy(
                tab_v.at[buf, :, pl.ds(0, S)],
                out_hbm.at[h, pl.ds(0, 8), :],
                sem,
            ).wait()

        def fire_class(buf, u):
            def fire(j, carry):
                blk = (15 - ncls * part - u) + 16 * j
                f = pl.multiple_of(1920 - 128 * j, 128)
                rs = pl.multiple_of(8 * blk, 8)
                pltpu.make_async_copy(
                    tab_v.at[buf, :, pl.ds(f, S)],
                    out_hbm.at[h, pl.ds(rs, 8), :],
                    sem,
                ).start()
                return carry

            lax.fori_loop(0, 16, fire, 0)

        def build_class(buf, parity, up):
            nw_regs = 2 if parity == 0 else 3

            def build(k, carry):
                base = -16 * up - 8 * ncls * part - 16 * k
                a_top = pl.multiple_of(base + 6128, 16)
                w = [t_v[pl.ds(a_top - 16 * n, 16)] for n in range(nw_regs)]
                for r in range(8):
                    cs = (7 + r - 8 * parity) % 16
                    ar = 6135 + r - 8 * parity - cs
                    rel = (6128 - ar) // 16
                    assert ar + 16 * rel == 6128 and rel + 1 < nw_regs
                    idx = (cs - lanes) & 15
                    mask = lanes <= cs
                    chunk = jnp.where(
                        mask, lane_perm(w[rel], idx), lane_perm(w[rel + 1], idx)
                    )
                    tab_v[buf, r, pl.ds(pl.multiple_of(16 * k, 16), 16)] = chunk
                return carry

            lax.fori_loop(0, COPY // 16, build, 0)

        def pair(up, carry):
            for parity in range(2):
                buf = parity

                @pl.when(up > 0)
                def _drain(buf=buf):
                    def d(j, c):
                        drain_one(buf)
                        return c

                    lax.fori_loop(0, 16, d, 0)

                build_class(buf, parity, up)
                fire_class(buf, 2 * up + parity)
            return carry

        lax.fori_loop(0, npair, pair, 0)

        def final_drain(j, carry):
            drain_one(0)
            drain_one(1)
            return carry

        lax.fori_loop(0, 16, final_drain, 0)

    return _rel_pos_bias


_half = _make(8)


def kernel(rel_bias, seq_len):
    del seq_len
    tbl = rel_bias[0]
    lo = _half(tbl[:8])
    hi = _half(tbl[8:])
    return jnp.concatenate([lo, hi], axis=0)[None]


# speedup vs baseline: 2.6814x; 2.6814x over previous
import functools

import jax
import jax.numpy as jnp
from jax import lax
from jax.experimental import pallas as pl
from jax.experimental.pallas import tpu as pltpu
from jax.experimental.pallas import tpu_sc as plsc

H = 16
S = 2048
TBL = 8191
COPY = 4096


_mesh = plsc.VectorSubcoreMesh(core_axis_name="c", subcore_axis_name="s")


@functools.partial(
    pl.kernel,
    out_type=jax.ShapeDtypeStruct((H, S, S), jnp.float32),
    mesh=_mesh,
    scratch_types=[
        pltpu.VMEM((TBL,), jnp.float32),
        pltpu.VMEM((2, 8, COPY), jnp.float32),
        pltpu.SemaphoreType.DMA,
    ],
)
def _rel_pos_bias(tbl_hbm, out_hbm, t_v, tab_v, sem):
    cid = lax.axis_index("c")
    sid = lax.axis_index("s")
    wid = sid * 2 + cid
    h = wid // 2
    chalf = wid % 2

    pltpu.sync_copy(tbl_hbm.at[h], t_v)

    lanes = lax.iota(jnp.int32, 16)

    def lane_perm(v, idx):
        dnums = lax.GatherDimensionNumbers(
            offset_dims=(), collapsed_slice_dims=(0,), start_index_map=(0,)
        )
        return lax.gather(
            v, idx[:, None], dnums, (1,),
            mode=lax.GatherScatterMode.PROMISE_IN_BOUNDS,
        )

    def drain_one(buf):
        pltpu.make_async_copy(
            tab_v.at[buf, :, pl.ds(0, S)],
            out_hbm.at[h, pl.ds(0, 8), :],
            sem,
        ).wait()

    def fire_class(buf, u):
        def fire(j, carry):
            blk = (15 - 8 * chalf - u) + 16 * j
            f = pl.multiple_of(1920 - 128 * j, 128)
            rs = pl.multiple_of(8 * blk, 8)
            pltpu.make_async_copy(
                tab_v.at[buf, :, pl.ds(f, S)],
                out_hbm.at[h, pl.ds(rs, 8), :],
                sem,
            ).start()
            return carry

        lax.fori_loop(0, 16, fire, 0)

    def build_class(buf, parity, up):
        nw = 2 if parity == 0 else 3

        def build(k, carry):
            base = -16 * up - 64 * chalf - 16 * k
            a_top = pl.multiple_of(base + 6128, 16)
            w = [t_v[pl.ds(a_top - 16 * n, 16)] for n in range(nw)]
            for r in range(8):
                cs = (7 + r - 8 * parity) % 16
                ar = 6135 + r - 8 * parity - cs
                rel = (6128 - ar) // 16
                assert ar + 16 * rel == 6128 and rel + 1 < nw, (r, parity)
                idx = (cs - lanes) & 15
                mask = lanes <= cs
                chunk = jnp.where(
                    mask, lane_perm(w[rel], idx), lane_perm(w[rel + 1], idx)
                )
                tab_v[buf, r, pl.ds(pl.multiple_of(16 * k, 16), 16)] = chunk
            return carry

        lax.fori_loop(0, COPY // 16, build, 0)

    def pair(up, carry):
        for parity in range(2):
            buf = parity

            @pl.when(up > 0)
            def _drain(buf=buf):
                def d(j, c):
                    drain_one(buf)
                    return c

                lax.fori_loop(0, 16, d, 0)

            build_class(buf, parity, up)
            fire_class(buf, 2 * up + parity)
        return carry

    lax.fori_loop(0, 4, pair, 0)

    def final_drain(j, carry):
        drain_one(0)
        drain_one(1)
        return carry

    lax.fori_loop(0, 16, final_drain, 0)


def kernel(rel_bias, seq_len):
    del seq_len
    return _rel_pos_bias(rel_bias[0])[None]
